# baseline (speedup 1.0000x reference)
import functools

import jax
import jax.numpy as jnp
from jax import lax
from jax.experimental import pallas as pl
from jax.experimental.pallas import tpu as pltpu
from jax.experimental.pallas import tpu_sc as plsc

NC = 2
NS = 16
NW = NC * NS
L = 16

NM, NU = 20000, 5000
NM_PAD, NU_PAD = 20480, 8192
CM, CU = NM_PAD // NW, NU_PAD // NW
VM, VU = CM // L, CU // L
RUN = 4 * CM + 4 * CU
GRP = 2 * CM + 2 * CU

_F = jnp.float32
_I = jnp.int32


def _build():
  mesh = plsc.VectorSubcoreMesh(
      core_axis_name="c", subcore_axis_name="s",
      num_cores=NC, num_subcores=NS)

  @functools.partial(
      pl.kernel,
      out_type=jax.ShapeDtypeStruct((NW, L), _F),
      mesh=mesh,
      scratch_types=[pltpu.VMEM((RUN,), _I),
                     pltpu.VMEM((GRP,), _F), pltpu.VMEM((GRP,), _F),
                     pltpu.VMEM((L,), _F), pltpu.SemaphoreType.DMA,
                     pltpu.SemaphoreType.DMA],
  )
  def run(pred_hbm, tgt_hbm, mi_hbm, ui_hbm, out_hbm,
          civ, vp, vt, acc_v, sem, gsem):
    wid = lax.axis_index("s") * NC + lax.axis_index("c")
    lanes = lax.iota(_I, L)

    jobs = []
    for l, (tab, vv) in enumerate([(pred_hbm, vp), (pred_hbm, vp),
                                   (tgt_hbm, vt), (tgt_hbm, vt)]):
      jobs.append((mi_hbm, l * NM_PAD + wid * CM, l * CM, CM,
                   tab, vv, (l % 2) * CM))
    for l, (tab, vv) in enumerate([(pred_hbm, vp), (pred_hbm, vp),
                                   (tgt_hbm, vt), (tgt_hbm, vt)]):
      jobs.append((ui_hbm, l * NU_PAD + wid * CU, 4 * CM + l * CU, CU,
                   tab, vv, 2 * CM + (l % 2) * CU))

    cps = [pltpu.async_copy(src.at[pl.ds(soff, sz)],
                            civ.at[pl.ds(coff, sz)], sem)
           for src, soff, coff, sz, _, _, _ in jobs]
    gps = []
    for cp, (_, _, coff, sz, tab, vv, voff) in zip(cps, jobs):
      cp.wait()
      gps.append(pltpu.async_copy(tab.at[civ.at[pl.ds(coff, sz)]],
                                  vv.at[pl.ds(voff, sz)], gsem))
    for g in gps:
      g.wait()

    def term(va, oa, vb, ob, nvec, ch, n_real):
      base = wid * ch
      def body(j, acc):
        o = j * L
        d = va[pl.ds(oa + o, L)] - vb[pl.ds(ob + o, L)]
        pos = base + o + lanes
        return acc + jnp.where(pos < n_real, d * d, jnp.zeros_like(d))
      return lax.fori_loop(0, nvec, body, jnp.zeros((L,), _F), unroll=4)

    t_b = term(vp, 0, vt, 0, VM, CM, NM)
    t_d = term(vp, CM, vt, CM, VM, CM, NM)
    t_up = term(vp, 2 * CM, vp, 2 * CM + CU, VU, CU, NU)
    t_ut = term(vt, 2 * CM, vt, 2 * CM + CU, VU, CU, NU)
    acc_v[...] = 2.0 * (t_b + t_d) + (t_up + t_ut)
    pltpu.sync_copy(acc_v, out_hbm.at[wid])

  return run


_run = _build()

_LIN_W = jnp.array([[16384.0], [128.0], [1.0]], jnp.float32)


def _lin4(lists, npad):
  c = jnp.stack(lists)
  i = (c.astype(jnp.float32) @ _LIN_W)[..., 0].astype(jnp.int32)
  return jnp.pad(i, ((0, 0), (0, npad - i.shape[1]))).reshape(-1)


def kernel(pred_field, tgt_field,
           matched_pred_birth, matched_pred_death,
           matched_tgt_birth, matched_tgt_death,
           unmatched_pred_birth, unmatched_pred_death,
           unmatched_tgt_birth, unmatched_tgt_death):
  mi = _lin4([matched_pred_birth, matched_pred_death,
              matched_tgt_birth, matched_tgt_death], NM_PAD)
  ui = _lin4([unmatched_pred_birth, unmatched_pred_death,
              unmatched_tgt_birth, unmatched_tgt_death], NU_PAD)
  out = _run(pred_field.reshape(-1), tgt_field.reshape(-1), mi, ui)
  return jnp.sum(out).reshape(1)


# speedup vs baseline: 1.9220x; 1.9220x over previous
import functools

import jax
import jax.numpy as jnp
from jax import lax
from jax.experimental import pallas as pl
from jax.experimental.pallas import tpu as pltpu
from jax.experimental.pallas import tpu_sc as plsc

NC = 2
NS = 16
NW = NC * NS
L = 16

NM, NU = 20000, 5000
NM_PAD, NU_PAD = 20480, 5120
CM, CU = NM_PAD // NW, NU_PAD // NW
VM, VU = CM // L, CU // L
GRP = 2 * CM + 2 * CU
GRP_PAD = 1664
RUN = 2 * GRP_PAD

_F = jnp.float32
_I = jnp.int32


def _build():
  mesh = plsc.VectorSubcoreMesh(
      core_axis_name="c", subcore_axis_name="s",
      num_cores=NC, num_subcores=NS)

  @functools.partial(
      pl.kernel,
      out_type=jax.ShapeDtypeStruct((NW, L), _F),
      mesh=mesh,
      scratch_types=[pltpu.VMEM((RUN,), _I),
                     pltpu.VMEM((GRP_PAD,), _F), pltpu.VMEM((GRP_PAD,), _F),
                     pltpu.VMEM((L,), _F), pltpu.SemaphoreType.DMA],
  )
  def run(pred_hbm, tgt_hbm, civ_hbm, out_hbm, civ, vp, vt, acc_v, sem):
    wid = lax.axis_index("s") * NC + lax.axis_index("c")
    lanes = lax.iota(_I, L)

    pltpu.async_copy(civ_hbm.at[pl.ds(wid * RUN, RUN)], civ, sem).wait()
    gps = []
    for tab, vv, goff in ((pred_hbm, vp, 0), (tgt_hbm, vt, GRP_PAD)):
      for off, sz in ((0, CM), (CM, CM), (2 * CM, 2 * CU)):
        gps.append(pltpu.async_copy(
            tab.at[civ.at[pl.ds(goff + off, sz)]], vv.at[pl.ds(off, sz)], sem))
    for g in gps:
      g.wait()

    def term(va, oa, vb, ob, nvec, ch, n_real):
      base = wid * ch
      def body(j, acc):
        o = j * L
        d = va[pl.ds(oa + o, L)] - vb[pl.ds(ob + o, L)]
        pos = base + o + lanes
        return acc + jnp.where(pos < n_real, d * d, jnp.zeros_like(d))
      return lax.fori_loop(0, nvec, body, jnp.zeros((L,), _F), unroll=4)

    t_b = term(vp, 0, vt, 0, VM, CM, NM)
    t_d = term(vp, CM, vt, CM, VM, CM, NM)
    t_up = term(vp, 2 * CM, vp, 2 * CM + CU, VU, CU, NU)
    t_ut = term(vt, 2 * CM, vt, 2 * CM + CU, VU, CU, NU)
    acc_v[...] = 2.0 * (t_b + t_d) + (t_up + t_ut)
    pltpu.sync_copy(acc_v, out_hbm.at[wid])

  return run


_run = _build()

_LIN_W = jnp.array([[16384.0], [128.0], [1.0]], jnp.float32)


def _lin4(lists, npad):
  c = jnp.stack(lists)
  i = (c.astype(jnp.float32) @ _LIN_W)[..., 0].astype(jnp.int32)
  i = jnp.pad(i, ((0, 0), (0, npad - i.shape[1])))
  ch = npad // NW
  return i.reshape(2, 2, NW, ch).transpose(2, 0, 1, 3).reshape(NW, 2, 2 * ch)


def kernel(pred_field, tgt_field,
           matched_pred_birth, matched_pred_death,
           matched_tgt_birth, matched_tgt_death,
           unmatched_pred_birth, unmatched_pred_death,
           unmatched_tgt_birth, unmatched_tgt_death):
  mi = _lin4([matched_pred_birth, matched_pred_death,
              matched_tgt_birth, matched_tgt_death], NM_PAD)
  ui = _lin4([unmatched_pred_birth, unmatched_pred_death,
              unmatched_tgt_birth, unmatched_tgt_death], NU_PAD)
  z = jnp.zeros((NW, 2, GRP_PAD - GRP), _I)
  civ = jnp.concatenate([mi, ui, z], axis=2).reshape(-1)
  out = _run(pred_field.reshape(-1), tgt_field.reshape(-1), civ)
  return jnp.sum(out).reshape(1)


# speedup vs baseline: 1.9871x; 1.0339x over previous
import functools

import jax
import jax.numpy as jnp
from jax import lax
from jax.experimental import pallas as pl
from jax.experimental.pallas import tpu as pltpu
from jax.experimental.pallas import tpu_sc as plsc

NC = 1
NS = 16
NW = NC * NS
L = 16

NM, NU = 20000, 5000
NM_PAD, NU_PAD = 20480, 5120
CM, CU = NM_PAD // NW, NU_PAD // NW
VM, VU = CM // L, CU // L
GRP = 2 * CM + 2 * CU
GRP_PAD = -(-GRP // 128) * 128
RUN = 2 * GRP_PAD

_F = jnp.float32
_I = jnp.int32


def _build():
  mesh = plsc.VectorSubcoreMesh(
      core_axis_name="c", subcore_axis_name="s",
      num_cores=NC, num_subcores=NS)

  @functools.partial(
      pl.kernel,
      out_type=jax.ShapeDtypeStruct((NW, L), _F),
      mesh=mesh,
      scratch_types=[pltpu.VMEM((RUN,), _I),
                     pltpu.VMEM((GRP_PAD,), _F), pltpu.VMEM((GRP_PAD,), _F),
                     pltpu.VMEM((L,), _F), pltpu.SemaphoreType.DMA],
  )
  def run(pred_hbm, tgt_hbm, civ_hbm, out_hbm, civ, vp, vt, acc_v, sem):
    wid = lax.axis_index("s") * NC + lax.axis_index("c")
    lanes = lax.iota(_I, L)

    pltpu.async_copy(civ_hbm.at[pl.ds(wid * RUN, RUN)], civ, sem).wait()
    gps = []
    for tab, vv, goff in ((pred_hbm, vp, 0), (tgt_hbm, vt, GRP_PAD)):
      for off, sz in ((0, CM), (CM, CM), (2 * CM, 2 * CU)):
        gps.append(pltpu.async_copy(
            tab.at[civ.at[pl.ds(goff + off, sz)]], vv.at[pl.ds(off, sz)], sem))
    for g in gps:
      g.wait()

    def term(va, oa, vb, ob, nvec, ch, n_real):
      base = wid * ch
      def body(j, acc):
        o = j * L
        d = va[pl.ds(oa + o, L)] - vb[pl.ds(ob + o, L)]
        pos = base + o + lanes
        return acc + jnp.where(pos < n_real, d * d, jnp.zeros_like(d))
      return lax.fori_loop(0, nvec, body, jnp.zeros((L,), _F), unroll=4)

    t_b = term(vp, 0, vt, 0, VM, CM, NM)
    t_d = term(vp, CM, vt, CM, VM, CM, NM)
    t_up = term(vp, 2 * CM, vp, 2 * CM + CU, VU, CU, NU)
    t_ut = term(vt, 2 * CM, vt, 2 * CM + CU, VU, CU, NU)
    acc_v[...] = 2.0 * (t_b + t_d) + (t_up + t_ut)
    pltpu.sync_copy(acc_v, out_hbm.at[wid])

  return run


_run = _build()

_LIN_W = jnp.array([[16384.0], [128.0], [1.0]], jnp.float32)


def _lin4(lists, npad):
  c = jnp.stack(lists)
  i = (c.astype(jnp.float32) @ _LIN_W)[..., 0].astype(jnp.int32)
  i = jnp.pad(i, ((0, 0), (0, npad - i.shape[1])))
  ch = npad // NW
  return i.reshape(2, 2, NW, ch).transpose(2, 0, 1, 3).reshape(NW, 2, 2 * ch)


def kernel(pred_field, tgt_field,
           matched_pred_birth, matched_pred_death,
           matched_tgt_birth, matched_tgt_death,
           unmatched_pred_birth, unmatched_pred_death,
           unmatched_tgt_birth, unmatched_tgt_death):
  mi = _lin4([matched_pred_birth, matched_pred_death,
              matched_tgt_birth, matched_tgt_death], NM_PAD)
  ui = _lin4([unmatched_pred_birth, unmatched_pred_death,
              unmatched_tgt_birth, unmatched_tgt_death], NU_PAD)
  parts = [mi, ui]
  if GRP_PAD > GRP:
    parts.append(jnp.zeros((NW, 2, GRP_PAD - GRP), _I))
  civ = jnp.concatenate(parts, axis=2).reshape(-1)
  out = _run(pred_field.reshape(-1), tgt_field.reshape(-1), civ)
  return jnp.sum(out).reshape(1)


# speedup vs baseline: 1.9923x; 1.0026x over previous
import functools

import jax
import jax.numpy as jnp
from jax import lax
from jax.experimental import pallas as pl
from jax.experimental.pallas import tpu as pltpu
from jax.experimental.pallas import tpu_sc as plsc

NC = 1
NS = 16
NW = NC * NS
L = 16

NM, NU = 20000, 5000
NM_PAD, NU_PAD = 20480, 5120
CM, CU = NM_PAD // NW, NU_PAD // NW
VM, VU = CM // L, CU // L
GRP = 2 * CM + 2 * CU
GRP_PAD = -(-GRP // 128) * 128
RUN = 2 * GRP_PAD

_F = jnp.float32
_I = jnp.int32


def _build():
  mesh = plsc.VectorSubcoreMesh(
      core_axis_name="c", subcore_axis_name="s",
      num_cores=NC, num_subcores=NS)

  @functools.partial(
      pl.kernel,
      out_type=jax.ShapeDtypeStruct((NW, L), _F),
      mesh=mesh,
      scratch_types=[pltpu.VMEM((RUN,), _I),
                     pltpu.VMEM((GRP_PAD,), _F), pltpu.VMEM((GRP_PAD,), _F),
                     pltpu.VMEM((L,), _F), pltpu.SemaphoreType.DMA],
  )
  def run(pred_hbm, tgt_hbm, civ_hbm, out_hbm, civ, vp, vt, acc_v, sem):
    wid = lax.axis_index("s") * NC + lax.axis_index("c")
    lanes = lax.iota(_I, L)

    pltpu.async_copy(civ_hbm.at[pl.ds(wid * RUN, RUN)], civ, sem).wait()
    gps = []
    half = CM // 2
    for tab, vv, goff in ((pred_hbm, vp, 0), (tgt_hbm, vt, GRP_PAD)):
      for off, sz in ((0, half), (half, half), (CM, half), (CM + half, half),
                      (2 * CM, 2 * CU)):
        gps.append(pltpu.async_copy(
            tab.at[civ.at[pl.ds(goff + off, sz)]], vv.at[pl.ds(off, sz)], sem))
    for g in gps:
      g.wait()

    def term(va, oa, vb, ob, nvec, ch, n_real):
      base = wid * ch
      def body(j, acc):
        o = j * L
        d = va[pl.ds(oa + o, L)] - vb[pl.ds(ob + o, L)]
        pos = base + o + lanes
        return acc + jnp.where(pos < n_real, d * d, jnp.zeros_like(d))
      return lax.fori_loop(0, nvec, body, jnp.zeros((L,), _F), unroll=4)

    t_b = term(vp, 0, vt, 0, VM, CM, NM)
    t_d = term(vp, CM, vt, CM, VM, CM, NM)
    t_up = term(vp, 2 * CM, vp, 2 * CM + CU, VU, CU, NU)
    t_ut = term(vt, 2 * CM, vt, 2 * CM + CU, VU, CU, NU)
    acc_v[...] = 2.0 * (t_b + t_d) + (t_up + t_ut)
    pltpu.sync_copy(acc_v, out_hbm.at[wid])

  return run


_run = _build()

_LIN_W = jnp.array([[16384.0], [128.0], [1.0]], jnp.float32)


def _lin4(lists, npad):
  c = jnp.stack(lists)
  i = (c.astype(jnp.float32) @ _LIN_W)[..., 0].astype(jnp.int32)
  i = jnp.pad(i, ((0, 0), (0, npad - i.shape[1])))
  ch = npad // NW
  return i.reshape(2, 2, NW, ch).transpose(2, 0, 1, 3).reshape(NW, 2, 2 * ch)


def kernel(pred_field, tgt_field,
           matched_pred_birth, matched_pred_death,
           matched_tgt_birth, matched_tgt_death,
           unmatched_pred_birth, unmatched_pred_death,
           unmatched_tgt_birth, unmatched_tgt_death):
  mi = _lin4([matched_pred_birth, matched_pred_death,
              matched_tgt_birth, matched_tgt_death], NM_PAD)
  ui = _lin4([unmatched_pred_birth, unmatched_pred_death,
              unmatched_tgt_birth, unmatched_tgt_death], NU_PAD)
  parts = [mi, ui]
  if GRP_PAD > GRP:
    parts.append(jnp.zeros((NW, 2, GRP_PAD - GRP), _I))
  civ = jnp.concatenate(parts, axis=2).reshape(-1)
  out = _run(pred_field.reshape(-1), tgt_field.reshape(-1), civ)
  return jnp.sum(out).reshape(1)


# speedup vs baseline: 2.0190x; 1.0134x over previous
import functools

import jax
import jax.numpy as jnp
from jax import lax
from jax.experimental import pallas as pl
from jax.experimental.pallas import tpu as pltpu
from jax.experimental.pallas import tpu_sc as plsc

NC = 1
NS = 16
NW = NC * NS
L = 16

NM, NU = 20000, 5000
NM_PAD, NU_PAD = 20480, 5120
CM, CU = NM_PAD // NW, NU_PAD // NW
VM, VU = CM // L, CU // L
RUNM, RUNU = 4 * CM, 4 * CU
GRP = 2 * CM + 2 * CU

_F = jnp.float32
_I = jnp.int32


def _build():
  mesh = plsc.VectorSubcoreMesh(
      core_axis_name="c", subcore_axis_name="s",
      num_cores=NC, num_subcores=NS)

  @functools.partial(
      pl.kernel,
      out_type=jax.ShapeDtypeStruct((NW, L), _F),
      mesh=mesh,
      scratch_types=[pltpu.VMEM((RUNM,), _I), pltpu.VMEM((RUNU,), _I),
                     pltpu.VMEM((GRP,), _F), pltpu.VMEM((GRP,), _F),
                     pltpu.VMEM((L,), _F), pltpu.SemaphoreType.DMA],
  )
  def run(pred_hbm, tgt_hbm, mi_hbm, ui_hbm, out_hbm,
          civm, civu, vp, vt, acc_v, sem):
    wid = lax.axis_index("s") * NC + lax.axis_index("c")
    lanes = lax.iota(_I, L)

    cpm = pltpu.async_copy(mi_hbm.at[pl.ds(wid * RUNM, RUNM)], civm, sem)
    cpu = pltpu.async_copy(ui_hbm.at[pl.ds(wid * RUNU, RUNU)], civu, sem)

    gps = []
    cpm.wait()
    for g, (tab, vv) in enumerate(((pred_hbm, vp), (tgt_hbm, vt))):
      for k in range(CM // 320):
        off = k * 640
        gps.append(pltpu.async_copy(
            tab.at[civm.at[pl.ds(g * 2 * CM + off, 640)]],
            vv.at[pl.ds(off, 640)], sem))
    cpu.wait()
    for g, (tab, vv) in enumerate(((pred_hbm, vp), (tgt_hbm, vt))):
      gps.append(pltpu.async_copy(
          tab.at[civu.at[pl.ds(g * 2 * CU, 2 * CU)]],
          vv.at[pl.ds(2 * CM, 2 * CU)], sem))
    for g in gps:
      g.wait()

    def term(va, oa, vb, ob, nvec, ch, n_real):
      base = wid * ch
      def body(j, acc):
        o = j * L
        d = va[pl.ds(oa + o, L)] - vb[pl.ds(ob + o, L)]
        pos = base + o + lanes
        return acc + jnp.where(pos < n_real, d * d, jnp.zeros_like(d))
      return lax.fori_loop(0, nvec, body, jnp.zeros((L,), _F), unroll=4)

    t_b = term(vp, 0, vt, 0, VM, CM, NM)
    t_d = term(vp, CM, vt, CM, VM, CM, NM)
    t_up = term(vp, 2 * CM, vp, 2 * CM + CU, VU, CU, NU)
    t_ut = term(vt, 2 * CM, vt, 2 * CM + CU, VU, CU, NU)
    acc_v[...] = 2.0 * (t_b + t_d) + (t_up + t_ut)
    pltpu.sync_copy(acc_v, out_hbm.at[wid])

  return run


_run = _build()

_LIN_W = jnp.array([[16384.0], [128.0], [1.0]], jnp.float32)


def _lin4(lists, npad):
  c = jnp.stack(lists)
  i = (c.astype(jnp.float32) @ _LIN_W)[..., 0].astype(jnp.int32)
  i = jnp.pad(i, ((0, 0), (0, npad - i.shape[1])))
  ch = npad // NW
  return i.reshape(2, 2, NW, ch).transpose(2, 0, 1, 3).reshape(-1)


def kernel(pred_field, tgt_field,
           matched_pred_birth, matched_pred_death,
           matched_tgt_birth, matched_tgt_death,
           unmatched_pred_birth, unmatched_pred_death,
           unmatched_tgt_birth, unmatched_tgt_death):
  mi = _lin4([matched_pred_birth, matched_pred_death,
              matched_tgt_birth, matched_tgt_death], NM_PAD)
  ui = _lin4([unmatched_pred_birth, unmatched_pred_death,
              unmatched_tgt_birth, unmatched_tgt_death], NU_PAD)
  out = _run(pred_field.reshape(-1), tgt_field.reshape(-1), mi, ui)
  return jnp.sum(out).reshape(1)
